# baseline (speedup 1.0000x reference)
import functools

import jax
import jax.numpy as jnp
from jax import lax
from jax.experimental import pallas as pl
from jax.experimental.pallas import tpu as pltpu
from jax.experimental.pallas import tpu_sc as plsc

L = 16
C = 128


@functools.cache
def _make_sc_loss(B, S, V):
    N = B * S
    n_dma = max(1, N // C)
    scale = -1.0 / B

    mesh = plsc.VectorSubcoreMesh(core_axis_name="c", subcore_axis_name="s")

    @functools.partial(
        pl.kernel,
        mesh=mesh,
        out_type=jax.ShapeDtypeStruct((L,), jnp.float32),
        scratch_types=[
            pltpu.VMEM((N,), jnp.int32),
            pltpu.VMEM((N,), jnp.float32),
            pltpu.VMEM((N,), jnp.int32),
            pltpu.VMEM((N,), jnp.float32),
            pltpu.VMEM((L,), jnp.float32),
            pltpu.SemaphoreType.DMA,
        ],
    )
    def sc_loss(inp_hbm, tgt_hbm, msk_hbm, out_hbm,
                tgt_v, msk_v, idx_v, val_v, stage_v, sem):
        c = lax.axis_index("c")
        s = lax.axis_index("s")

        @pl.when(jnp.logical_and(c == 0, s == 0))
        def _work():
            pltpu.sync_copy(tgt_hbm, tgt_v)
            pltpu.sync_copy(msk_hbm, msk_v)
            for j in range(N // L):
                t = tgt_v[pl.ds(j * L, L)]
                pos = (j * L) + lax.iota(jnp.int32, L)
                idx_v[pl.ds(j * L, L)] = pos * V + t
            copies = [
                pltpu.async_copy(
                    inp_hbm.at[idx_v.at[pl.ds(k * C, C)]],
                    val_v.at[pl.ds(k * C, C)],
                    sem,
                )
                for k in range(n_dma)
            ]
            for cp in copies:
                cp.wait()
            acc = jnp.zeros((L,), jnp.float32)
            for j in range(N // L):
                acc = acc + val_v[pl.ds(j * L, L)] * msk_v[pl.ds(j * L, L)]
            total = acc[0]
            for i in range(1, L):
                total = total + acc[i]
            total = total * scale
            stage_v[...] = jnp.broadcast_to(total, (L,))
            pltpu.sync_copy(stage_v, out_hbm)

    return sc_loss


def kernel(input, target, mask):
    B, S, V = input.shape
    inp_flat = input.reshape(B * S * V)
    tgt = target.reshape(-1).astype(jnp.int32)
    msk = mask.reshape(-1).astype(jnp.float32)
    out = _make_sc_loss(B, S, V)(inp_flat, tgt, msk)
    return out[0]


# speedup vs baseline: 11.7948x; 11.7948x over previous
import functools

import jax
import jax.numpy as jnp
from jax import lax
from jax.experimental import pallas as pl
from jax.experimental.pallas import tpu as pltpu
from jax.experimental.pallas import tpu_sc as plsc

L = 16
SUB, LAN = 8, 128


@functools.cache
def _make_sc_loss(B, S, V):
    scale = -1.0 / B

    mesh = plsc.VectorSubcoreMesh(core_axis_name="c", subcore_axis_name="s")

    @functools.partial(
        pl.kernel,
        mesh=mesh,
        out_type=jax.ShapeDtypeStruct((B, 1, L), jnp.float32),
        scratch_types=[
            pltpu.VMEM((B, S), jnp.int32),
            pltpu.VMEM((B, S), jnp.float32),
            pltpu.VMEM((S, SUB, LAN), jnp.float32),
            pltpu.VMEM((L,), jnp.float32),
            pltpu.SemaphoreType.DMA,
        ],
    )
    def sc_loss(inp_hbm, tgt_hbm, msk_hbm, out_hbm,
                tgt_v, msk_v, blk_v, stage_v, sem):
        c = lax.axis_index("c")
        s = lax.axis_index("s")
        wid = c * 16 + s

        pltpu.sync_copy(tgt_hbm, tgt_v)
        pltpu.sync_copy(msk_hbm, msk_v)
        tv = tgt_v[wid, :]
        mv = msk_v[wid, :]

        for r in range(S):
            t = tv[r]
            t0 = (t // LAN) * LAN
            q0 = (r // SUB) * SUB
            pltpu.async_copy(
                inp_hbm.at[wid, pl.ds(q0, SUB), pl.ds(t0, LAN)],
                blk_v.at[r], sem,
            )
        for r in range(S):
            pltpu.make_async_copy(
                inp_hbm.at[0, pl.ds(0, SUB), pl.ds(0, LAN)],
                blk_v.at[0], sem,
            ).wait()

        iota = lax.iota(jnp.int32, L)
        acc = jnp.zeros((L,), jnp.float32)
        for r in range(S):
            t = tv[r]
            lrem = t - (t // LAN) * LAN
            mk = mv[r]
            for h in range(LAN // L):
                rv = blk_v[r, r % SUB, pl.ds(h * L, L)]
                acc = acc + jnp.where(iota + (h * L) == lrem, rv * mk, 0.0)

        total = acc[0]
        for i in range(1, L):
            total = total + acc[i]
        total = total * scale
        stage_v[...] = jnp.broadcast_to(total, (L,))
        pltpu.sync_copy(stage_v, out_hbm.at[wid, 0])

    return sc_loss


def kernel(input, target, mask):
    B, S, V = input.shape
    tgt = target.astype(jnp.int32)
    msk = mask.astype(jnp.float32)
    parts = _make_sc_loss(B, S, V)(input, tgt, msk)
    return jnp.sum(parts[:, 0, 0])


# speedup vs baseline: 12.4093x; 1.0521x over previous
import functools

import jax
import jax.numpy as jnp
from jax import lax
from jax.experimental import pallas as pl
from jax.experimental.pallas import tpu as pltpu
from jax.experimental.pallas import tpu_sc as plsc

L = 16
SUB, LAN = 8, 128


@functools.cache
def _make_sc_loss(B, S, V):
    scale = -1.0 / B

    mesh = plsc.VectorSubcoreMesh(core_axis_name="c", subcore_axis_name="s")

    @functools.partial(
        pl.kernel,
        mesh=mesh,
        out_type=jax.ShapeDtypeStruct((B, 1, L), jnp.float32),
        scratch_types=[
            pltpu.VMEM((SUB, S), jnp.int32),
            pltpu.VMEM((SUB, S), jnp.float32),
            pltpu.VMEM((S, SUB, LAN), jnp.float32),
            pltpu.VMEM((L,), jnp.float32),
            pltpu.SemaphoreType.DMA,
        ],
    )
    def sc_loss(inp_hbm, tgt_hbm, msk_hbm, out_hbm,
                tgt_v, msk_v, blk_v, stage_v, sem):
        c = lax.axis_index("c")
        s = lax.axis_index("s")
        wid = c * 16 + s

        w0 = (wid // SUB) * SUB
        woff = wid - w0
        pltpu.sync_copy(tgt_hbm.at[pl.ds(w0, SUB)], tgt_v)
        pltpu.sync_copy(msk_hbm.at[pl.ds(w0, SUB)], msk_v)
        tv = tgt_v[woff, :]
        mv = msk_v[woff, :]

        for r in range(S):
            t = tv[r]
            t0 = (t // LAN) * LAN
            q0 = (r // SUB) * SUB
            pltpu.async_copy(
                inp_hbm.at[wid, pl.ds(q0, SUB), pl.ds(t0, LAN)],
                blk_v.at[r], sem,
            )
        for r in range(S):
            pltpu.make_async_copy(
                inp_hbm.at[0, pl.ds(0, SUB), pl.ds(0, LAN)],
                blk_v.at[0], sem,
            ).wait()

        iota = lax.iota(jnp.int32, L)
        acc = jnp.zeros((L,), jnp.float32)
        for r in range(S):
            t = tv[r]
            lrem = t - (t // LAN) * LAN
            mk = mv[r]
            for h in range(LAN // L):
                rv = blk_v[r, r % SUB, pl.ds(h * L, L)]
                acc = acc + jnp.where(iota + (h * L) == lrem, rv * mk, 0.0)

        total = acc[0]
        for i in range(1, L):
            total = total + acc[i]
        total = total * scale
        stage_v[...] = jnp.broadcast_to(total, (L,))
        pltpu.sync_copy(stage_v, out_hbm.at[wid, 0])

    return sc_loss


def kernel(input, target, mask):
    B, S, V = input.shape
    tgt = target.astype(jnp.int32)
    msk = mask.astype(jnp.float32)
    parts = _make_sc_loss(B, S, V)(input, tgt, msk)
    return jnp.sum(parts[:, 0, 0])


# speedup vs baseline: 12.6429x; 1.0188x over previous
import functools

import jax
import jax.numpy as jnp
from jax import lax
from jax.experimental import pallas as pl
from jax.experimental.pallas import tpu as pltpu
from jax.experimental.pallas import tpu_sc as plsc

L = 16
SUB, LAN = 8, 128


@functools.cache
def _make_sc_loss(B, S, V):
    scale = -1.0 / B

    mesh = plsc.VectorSubcoreMesh(core_axis_name="c", subcore_axis_name="s")

    @functools.partial(
        pl.kernel,
        mesh=mesh,
        out_type=jax.ShapeDtypeStruct((B, 1, L), jnp.float32),
        scratch_types=[
            pltpu.VMEM((SUB, S), jnp.int32),
            pltpu.VMEM((SUB, S), jnp.float32),
            pltpu.VMEM((S, SUB, LAN), jnp.float32),
            pltpu.VMEM((L,), jnp.float32),
            pltpu.SemaphoreType.DMA,
            pltpu.SemaphoreType.DMA,
        ],
    )
    def sc_loss(inp_hbm, tgt_hbm, msk_hbm, out_hbm,
                tgt_v, msk_v, blk_v, stage_v, sem, sem2):
        c = lax.axis_index("c")
        s = lax.axis_index("s")
        wid = c * 16 + s

        w0 = (wid // SUB) * SUB
        woff = wid - w0
        msk_cp = pltpu.async_copy(msk_hbm.at[pl.ds(w0, SUB)], msk_v, sem2)
        pltpu.sync_copy(tgt_hbm.at[pl.ds(w0, SUB)], tgt_v)
        tv = tgt_v[woff, :]

        for r in range(S):
            t = tv[r]
            t0 = (t // LAN) * LAN
            q0 = (r // SUB) * SUB
            pltpu.async_copy(
                inp_hbm.at[wid, pl.ds(q0, SUB), pl.ds(t0, LAN)],
                blk_v.at[r], sem,
            )
        for r in range(S):
            pltpu.make_async_copy(
                inp_hbm.at[0, pl.ds(0, SUB), pl.ds(0, LAN)],
                blk_v.at[0], sem,
            ).wait()

        msk_cp.wait()
        mv = msk_v[woff, :]
        iota = lax.iota(jnp.int32, L)
        acc = jnp.zeros((L,), jnp.float32)
        for r in range(S):
            t = tv[r]
            lrem = t - (t // LAN) * LAN
            mk = mv[r]
            for h in range(LAN // L):
                rv = blk_v[r, r % SUB, pl.ds(h * L, L)]
                acc = acc + jnp.where(iota + (h * L) == lrem, rv * mk, 0.0)

        total = acc[0]
        for i in range(1, L):
            total = total + acc[i]
        total = total * scale
        stage_v[...] = jnp.broadcast_to(total, (L,))
        pltpu.sync_copy(stage_v, out_hbm.at[wid, 0])

    return sc_loss


def kernel(input, target, mask):
    B, S, V = input.shape
    tgt = target.astype(jnp.int32)
    msk = mask.astype(jnp.float32)
    parts = _make_sc_loss(B, S, V)(input, tgt, msk)
    return jnp.sum(parts[:, 0, 0])


# speedup vs baseline: 12.6741x; 1.0025x over previous
import functools

import jax
import jax.numpy as jnp
from jax import lax
from jax.experimental import pallas as pl
from jax.experimental.pallas import tpu as pltpu
from jax.experimental.pallas import tpu_sc as plsc

L = 16
SUB, LAN = 8, 128


@functools.cache
def _make_sc_loss(B, S, V):
    scale = -1.0 / B
    n_sub = 16
    rows = B // n_sub

    mesh = plsc.VectorSubcoreMesh(
        core_axis_name="c", subcore_axis_name="s", num_cores=1
    )

    @functools.partial(
        pl.kernel,
        mesh=mesh,
        out_type=jax.ShapeDtypeStruct((n_sub, 1, L), jnp.float32),
        scratch_types=[
            pltpu.VMEM((SUB, S), jnp.int32),
            pltpu.VMEM((SUB, S), jnp.float32),
            pltpu.VMEM((rows * S, SUB, LAN), jnp.float32),
            pltpu.VMEM((L,), jnp.float32),
            pltpu.SemaphoreType.DMA,
            pltpu.SemaphoreType.DMA,
        ],
    )
    def sc_loss(inp_hbm, tgt_hbm, msk_hbm, out_hbm,
                tgt_v, msk_v, blk_v, stage_v, sem, sem2):
        s = lax.axis_index("s")
        b_first = s * rows
        w0 = (b_first // SUB) * SUB
        woff = b_first - w0
        msk_cp = pltpu.async_copy(msk_hbm.at[pl.ds(w0, SUB)], msk_v, sem2)
        pltpu.sync_copy(tgt_hbm.at[pl.ds(w0, SUB)], tgt_v)
        tvs = [tgt_v[woff + j, :] for j in range(rows)]

        for j in range(rows):
            for r in range(S):
                t = tvs[j][r]
                t0 = (t // LAN) * LAN
                q0 = (r // SUB) * SUB
                pltpu.async_copy(
                    inp_hbm.at[b_first + j, pl.ds(q0, SUB), pl.ds(t0, LAN)],
                    blk_v.at[j * S + r], sem,
                )
        for _ in range(rows * S):
            pltpu.make_async_copy(
                inp_hbm.at[0, pl.ds(0, SUB), pl.ds(0, LAN)],
                blk_v.at[0], sem,
            ).wait()

        msk_cp.wait()
        iota = lax.iota(jnp.int32, L)
        acc = jnp.zeros((L,), jnp.float32)
        for j in range(rows):
            mv = msk_v[woff + j, :]
            for r in range(S):
                t = tvs[j][r]
                lrem = t - (t // LAN) * LAN
                mk = mv[r]
                for h in range(LAN // L):
                    rv = blk_v[j * S + r, r % SUB, pl.ds(h * L, L)]
                    acc = acc + jnp.where(iota + (h * L) == lrem, rv * mk, 0.0)

        total = acc[0]
        for i in range(1, L):
            total = total + acc[i]
        total = total * scale
        stage_v[...] = jnp.broadcast_to(total, (L,))
        pltpu.sync_copy(stage_v, out_hbm.at[s, 0])

    return sc_loss


def kernel(input, target, mask):
    B, S, V = input.shape
    tgt = target.astype(jnp.int32)
    msk = mask.astype(jnp.float32)
    parts = _make_sc_loss(B, S, V)(input, tgt, msk)
    return jnp.sum(parts[:, 0, 0])
